# baseline (speedup 1.0000x reference)
import functools

import jax
import jax.numpy as jnp
from jax import lax
from jax.experimental import pallas as pl
from jax.experimental.pallas import tpu as pltpu
from jax.experimental.pallas import tpu_sc as plsc

U = 5000
D = 128
E = 320000
P = 32768
NLAYER = 3

NT = 16
NP = 5120
ROWS = NP // NT
EC = 128
NCH = 159
EPT = NCH * EC
EBLK = 32
SEC = 128

_mesh = plsc.VectorSubcoreMesh(core_axis_name="c", subcore_axis_name="s")


@functools.partial(
    pl.kernel,
    out_type=jax.ShapeDtypeStruct((2 * NP,), jnp.float32),
    mesh=_mesh,
    scratch_types=[
        pltpu.VMEM((NCH, EC), jnp.int32),
        pltpu.VMEM((EC,), jnp.float32),
        pltpu.VMEM((ROWS,), jnp.float32),
        pltpu.VMEM_SHARED((NP,), jnp.float32),
    ],
)
def _degree_kernel(idx_hbm, deg_hbm, idx_v, ones_v, stage_v, acc_sh):
    c = lax.axis_index("c")
    s = lax.axis_index("s")

    def fill(i, _):
        ones_v[pl.ds(i * 16, 16)] = jnp.ones((16,), jnp.float32)
        return 0
    lax.fori_loop(0, EC // 16, fill, 0)

    def zfill(i, _):
        stage_v[pl.ds(i * 16, 16)] = jnp.zeros((16,), jnp.float32)
        return 0
    lax.fori_loop(0, ROWS // 16, zfill, 0)
    pltpu.sync_copy(stage_v, acc_sh.at[pl.ds(s * ROWS, ROWS)])
    pltpu.sync_copy(idx_hbm.at[c, s], idx_v)
    plsc.subcore_barrier()

    def body(j, _):
        pltpu.sync_copy(ones_v, acc_sh.at[idx_v.at[j]], add=True)
        return 0
    lax.fori_loop(0, NCH, body, 0)

    plsc.subcore_barrier()
    pltpu.sync_copy(acc_sh.at[pl.ds(s * ROWS, ROWS)], stage_v)
    pltpu.sync_copy(stage_v, deg_hbm.at[pl.ds(c * NP + s * ROWS, ROWS)])


@functools.partial(
    pl.kernel,
    out_type=(jax.ShapeDtypeStruct((2, NP, D), jnp.float32),
              jax.ShapeDtypeStruct((2, NP, D), jnp.float32)),
    mesh=_mesh,
    scratch_types=[
        pltpu.VMEM((NCH, EC), jnp.int32),
        pltpu.VMEM((NCH, EC), jnp.int32),
        pltpu.VMEM((EC, D), jnp.float32),
        pltpu.VMEM((EC, D), jnp.float32),
        pltpu.VMEM((EC, D), jnp.float32),
        pltpu.VMEM_SHARED((NP, D), jnp.float32),
        pltpu.SemaphoreType.DMA,
        pltpu.SemaphoreType.DMA,
        pltpu.SemaphoreType.DMA,
        pltpu.SemaphoreType.DMA,
        pltpu.SemaphoreType.DMA,
        pltpu.SemaphoreType.DMA,
    ],
)
def _layer_kernel(g_hbm, hsum_hbm, idx_hbm, nsqb_hbm, nb_hbm, zeros_hbm,
                  gnext_hbm, hout_hbm,
                  gidx_v, sidx_v, bufa, bufb, bufc, acc_sh,
                  gsa, gsb, gsc, ssa, ssb, ssc):
    c = lax.axis_index("c")
    s = lax.axis_index("s")
    d = 1 - c

    pltpu.sync_copy(zeros_hbm.at[pl.ds(0, EBLK)], bufa.at[pl.ds(0, EBLK)])

    def zblk(b, _):
        pltpu.sync_copy(bufa.at[pl.ds(0, EBLK)],
                        acc_sh.at[pl.ds(s * ROWS + b * EBLK, EBLK)])
        return 0
    lax.fori_loop(0, ROWS // EBLK, zblk, 0)
    pltpu.sync_copy(idx_hbm.at[c, s], gidx_v)
    pltpu.sync_copy(idx_hbm.at[d, s], sidx_v)
    plsc.subcore_barrier()

    table = g_hbm.at[c]

    def g_of(kk):
        return gidx_v.at[jnp.where(kk < NCH, kk, 0)]

    pltpu.async_copy(table.at[gidx_v.at[0]], bufa, gsa)
    pltpu.async_copy(table.at[gidx_v.at[1]], bufb, gsb)

    def body(j, _):
        k0 = 3 * j
        pltpu.make_async_copy(table.at[g_of(k0)], bufa, gsa).wait()
        pltpu.async_copy(bufa, acc_sh.at[sidx_v.at[k0]], ssa, add=True)

        @pl.when(j >= 1)
        def _():
            pltpu.make_async_copy(bufc, acc_sh.at[sidx_v.at[k0 - 1]],
                                  ssc).wait()
        pltpu.async_copy(table.at[g_of(k0 + 2)], bufc, gsc)
        pltpu.make_async_copy(table.at[g_of(k0 + 1)], bufb, gsb).wait()
        pltpu.async_copy(bufb, acc_sh.at[sidx_v.at[k0 + 1]], ssb, add=True)
        pltpu.make_async_copy(bufa, acc_sh.at[sidx_v.at[k0]], ssa).wait()
        pltpu.async_copy(table.at[g_of(k0 + 3)], bufa, gsa)
        pltpu.make_async_copy(table.at[g_of(k0 + 2)], bufc, gsc).wait()
        pltpu.async_copy(bufc, acc_sh.at[sidx_v.at[k0 + 2]], ssc, add=True)
        pltpu.make_async_copy(bufb, acc_sh.at[sidx_v.at[k0 + 1]], ssb).wait()
        pltpu.async_copy(table.at[g_of(k0 + 4)], bufb, gsb)
        return 0
    lax.fori_loop(0, NCH // 3, body, 0)
    pltpu.make_async_copy(table.at[gidx_v.at[0]], bufa, gsa).wait()
    pltpu.make_async_copy(table.at[gidx_v.at[0]], bufb, gsb).wait()
    pltpu.make_async_copy(bufc, acc_sh.at[sidx_v.at[NCH - 1]], ssc).wait()

    plsc.subcore_barrier()

    eacc = bufa.at[pl.ds(0, EBLK)]
    escl = bufb.at[pl.ds(0, EBLK)]
    eres = bufc.at[pl.ds(0, EBLK)]

    def eblock(blk, _):
        r0 = s * ROWS + blk * EBLK
        pltpu.sync_copy(acc_sh.at[pl.ds(r0, EBLK)], eacc)
        pltpu.sync_copy(nsqb_hbm.at[d, pl.ds(r0, EBLK)], escl)

        def rows_g(r, _):
            for q in range(D // 16):
                sl = pl.ds(q * 16, 16)
                bufb[r, sl] = bufa[r, sl] * bufb[r, sl]
            return 0
        lax.fori_loop(0, EBLK, rows_g, 0)
        pltpu.sync_copy(escl, gnext_hbm.at[d, pl.ds(r0, EBLK)])

        pltpu.sync_copy(nb_hbm.at[d, pl.ds(r0, EBLK)], escl)
        pltpu.sync_copy(hsum_hbm.at[d, pl.ds(r0, EBLK)], eres)

        def rows_h(r, _):
            for q in range(D // 16):
                sl = pl.ds(q * 16, 16)
                bufc[r, sl] = bufc[r, sl] + bufa[r, sl] * bufb[r, sl]
            return 0
        lax.fori_loop(0, EBLK, rows_h, 0)
        pltpu.sync_copy(eres, hout_hbm.at[d, pl.ds(r0, EBLK)])
        return 0
    lax.fori_loop(0, ROWS // EBLK, eblock, 0)


@functools.partial(
    pl.kernel,
    out_type=jax.ShapeDtypeStruct((2, 2, P, D), jnp.float32),
    mesh=_mesh,
    scratch_types=[
        pltpu.VMEM((P // NT // SEC, SEC), jnp.int32),
        pltpu.VMEM((SEC, D), jnp.float32),
        pltpu.VMEM((SEC, D), jnp.float32),
        pltpu.SemaphoreType.DMA,
        pltpu.SemaphoreType.DMA,
    ],
)
def _score_gather_kernel(hsum_hbm, pidx_hbm, out_hbm, cidx_v, buf0, buf1,
                         semg0, semg1):
    c = lax.axis_index("c")
    s = lax.axis_index("s")
    ept = P // NT
    nch = ept // SEC

    for side in range(2):
        pltpu.sync_copy(pidx_hbm.at[c, side, s], cidx_v)
        table = hsum_hbm.at[side]
        out = out_hbm.at[c, side]
        pltpu.async_copy(table.at[cidx_v.at[0]], buf0, semg0)

        def body(j, _):
            k0 = 2 * j

            @pl.when(k0 + 1 < nch)
            def _():
                pltpu.async_copy(table.at[cidx_v.at[k0 + 1]], buf1, semg1)
            pltpu.make_async_copy(table.at[cidx_v.at[k0]], buf0, semg0).wait()
            pltpu.sync_copy(buf0, out.at[pl.ds(s * ept + k0 * SEC, SEC)])

            @pl.when(k0 + 2 < nch)
            def _():
                pltpu.async_copy(table.at[cidx_v.at[k0 + 2]], buf0, semg0)
            pltpu.make_async_copy(table.at[cidx_v.at[k0 + 1]], buf1,
                                  semg1).wait()
            pltpu.sync_copy(buf1, out.at[pl.ds(s * ept + k0 * SEC + SEC, SEC)])
            return 0
        lax.fori_loop(0, nch // 2, body, 0)


def _prep_body(deg_ref, emb_ref, g0_ref, nsqb_ref, nb_ref):
    dg = jnp.maximum(deg_ref[0], 1.0)
    norm = lax.rsqrt(dg)
    nsq = 1.0 / dg
    g0_ref[0] = emb_ref[0] * norm
    nsqb_ref[0] = jnp.broadcast_to(nsq, nsqb_ref.shape[1:])
    nb_ref[0] = jnp.broadcast_to(norm, nb_ref.shape[1:])


_BR = 512


def _prep(deg3, empad):
    return pl.pallas_call(
        _prep_body,
        grid=(2, NP // _BR),
        in_specs=[
            pl.BlockSpec((1, _BR, 1), lambda i, j: (i, j, 0)),
            pl.BlockSpec((1, _BR, D), lambda i, j: (i, j, 0)),
        ],
        out_specs=[
            pl.BlockSpec((1, _BR, D), lambda i, j: (i, j, 0)),
            pl.BlockSpec((1, _BR, D), lambda i, j: (i, j, 0)),
            pl.BlockSpec((1, _BR, D), lambda i, j: (i, j, 0)),
        ],
        out_shape=[
            jax.ShapeDtypeStruct((2, NP, D), jnp.float32),
            jax.ShapeDtypeStruct((2, NP, D), jnp.float32),
            jax.ShapeDtypeStruct((2, NP, D), jnp.float32),
        ],
    )(deg3, empad)


def _dot_body(a_ref, b_ref, out_ref):
    out_ref[0] = (float(NLAYER) * float(NLAYER)) * jnp.sum(
        a_ref[0, 0] * b_ref[0, 0], axis=-1, keepdims=True)


_BP = 1024


def _dots(ab):
    return pl.pallas_call(
        _dot_body,
        grid=(2, P // _BP),
        in_specs=[
            pl.BlockSpec((1, 1, _BP, D), lambda i, j: (i, 0, j, 0)),
            pl.BlockSpec((1, 1, _BP, D), lambda i, j: (i, 1, j, 0)),
        ],
        out_specs=pl.BlockSpec((1, _BP, 1), lambda i, j: (i, j, 0)),
        out_shape=jax.ShapeDtypeStruct((2, P, 1), jnp.float32),
    )(ab, ab)


def kernel(msg_edges, pos_edges, neg_edges, user_emb, item_emb):
    pad_n = NT * NCH * EC - E
    pad_idx = U + (jnp.arange(pad_n, dtype=jnp.int32) % (NP - U))
    idx = jnp.concatenate(
        [msg_edges.astype(jnp.int32),
         jnp.broadcast_to(pad_idx, (2, pad_n))], axis=1)
    idx = idx.reshape(2, NT, NCH, EC)

    deg = _degree_kernel(idx)

    empad = jnp.stack([
        jnp.pad(user_emb, ((0, NP - U), (0, 0))),
        jnp.pad(item_emb, ((0, NP - U), (0, 0))),
    ])
    g0, nsqb, nb = _prep(deg.reshape(2, NP, 1), empad)

    zeros = jnp.zeros((NP, D), jnp.float32)
    g, hsum = g0, empad
    for _ in range(NLAYER):
        g, hsum = _layer_kernel(g, hsum, idx, nsqb, nb, zeros)

    pidx = jnp.stack([pos_edges, neg_edges]).astype(jnp.int32)
    pidx = pidx.reshape(2, 2, NT, P // NT // SEC, SEC)
    ab = _score_gather_kernel(hsum, pidx)
    scores = _dots(ab)
    return scores[0], scores[1]


# speedup vs baseline: 1.0143x; 1.0143x over previous
import functools

import jax
import jax.numpy as jnp
from jax import lax
from jax.experimental import pallas as pl
from jax.experimental.pallas import tpu as pltpu
from jax.experimental.pallas import tpu_sc as plsc

U = 5000
D = 128
E = 320000
P = 32768
NLAYER = 3

NT = 16
NP = 5120
ROWS = NP // NT
EC = 112
NCH = 180
EPT = NCH * EC
EBLK = 32
SEC = 128

_mesh = plsc.VectorSubcoreMesh(core_axis_name="c", subcore_axis_name="s")


@functools.partial(
    pl.kernel,
    out_type=jax.ShapeDtypeStruct((2 * NP,), jnp.float32),
    mesh=_mesh,
    scratch_types=[
        pltpu.VMEM((NCH, EC), jnp.int32),
        pltpu.VMEM((EC,), jnp.float32),
        pltpu.VMEM((ROWS,), jnp.float32),
        pltpu.VMEM_SHARED((NP,), jnp.float32),
    ],
)
def _degree_kernel(idx_hbm, deg_hbm, idx_v, ones_v, stage_v, acc_sh):
    c = lax.axis_index("c")
    s = lax.axis_index("s")

    def fill(i, _):
        ones_v[pl.ds(i * 16, 16)] = jnp.ones((16,), jnp.float32)
        return 0
    lax.fori_loop(0, EC // 16, fill, 0)

    def zfill(i, _):
        stage_v[pl.ds(i * 16, 16)] = jnp.zeros((16,), jnp.float32)
        return 0
    lax.fori_loop(0, ROWS // 16, zfill, 0)
    pltpu.sync_copy(stage_v, acc_sh.at[pl.ds(s * ROWS, ROWS)])
    pltpu.sync_copy(idx_hbm.at[c, s], idx_v)
    plsc.subcore_barrier()

    def body(j, _):
        pltpu.sync_copy(ones_v, acc_sh.at[idx_v.at[j]], add=True)
        return 0
    lax.fori_loop(0, NCH, body, 0)

    plsc.subcore_barrier()
    pltpu.sync_copy(acc_sh.at[pl.ds(s * ROWS, ROWS)], stage_v)
    pltpu.sync_copy(stage_v, deg_hbm.at[pl.ds(c * NP + s * ROWS, ROWS)])


@functools.partial(
    pl.kernel,
    out_type=(jax.ShapeDtypeStruct((2, NP, D), jnp.float32),
              jax.ShapeDtypeStruct((2, NP, D), jnp.float32)),
    mesh=_mesh,
    scratch_types=[
        pltpu.VMEM((NCH, EC), jnp.int32),
        pltpu.VMEM((NCH, EC), jnp.int32),
        pltpu.VMEM((EC, D), jnp.float32),
        pltpu.VMEM((EC, D), jnp.float32),
        pltpu.VMEM((EC, D), jnp.float32),
        pltpu.VMEM_SHARED((NP, D), jnp.float32),
        pltpu.SemaphoreType.DMA,
        pltpu.SemaphoreType.DMA,
        pltpu.SemaphoreType.DMA,
        pltpu.SemaphoreType.DMA,
        pltpu.SemaphoreType.DMA,
        pltpu.SemaphoreType.DMA,
    ],
)
def _layer_kernel(g_hbm, hsum_hbm, idx_hbm, nsqb_hbm, nb_hbm, zeros_hbm,
                  gnext_hbm, hout_hbm,
                  gidx_v, sidx_v, bufa, bufb, bufc, acc_sh,
                  gsa, gsb, gsc, ssa, ssb, ssc):
    c = lax.axis_index("c")
    s = lax.axis_index("s")
    d = 1 - c

    pltpu.sync_copy(zeros_hbm.at[pl.ds(0, EBLK)], bufa.at[pl.ds(0, EBLK)])

    def zblk(b, _):
        pltpu.sync_copy(bufa.at[pl.ds(0, EBLK)],
                        acc_sh.at[pl.ds(s * ROWS + b * EBLK, EBLK)])
        return 0
    lax.fori_loop(0, ROWS // EBLK, zblk, 0)
    pltpu.sync_copy(idx_hbm.at[c, s], gidx_v)
    pltpu.sync_copy(idx_hbm.at[d, s], sidx_v)
    plsc.subcore_barrier()

    table = g_hbm.at[c]

    def g_of(kk):
        return gidx_v.at[jnp.where(kk < NCH, kk, 0)]

    pltpu.async_copy(table.at[gidx_v.at[0]], bufa, gsa)
    pltpu.async_copy(table.at[gidx_v.at[1]], bufb, gsb)

    def body(j, _):
        k0 = 3 * j
        pltpu.make_async_copy(table.at[g_of(k0)], bufa, gsa).wait()
        pltpu.async_copy(bufa, acc_sh.at[sidx_v.at[k0]], ssa, add=True)

        @pl.when(j >= 1)
        def _():
            pltpu.make_async_copy(bufc, acc_sh.at[sidx_v.at[k0 - 1]],
                                  ssc).wait()
        pltpu.async_copy(table.at[g_of(k0 + 2)], bufc, gsc)
        pltpu.make_async_copy(table.at[g_of(k0 + 1)], bufb, gsb).wait()
        pltpu.async_copy(bufb, acc_sh.at[sidx_v.at[k0 + 1]], ssb, add=True)
        pltpu.make_async_copy(bufa, acc_sh.at[sidx_v.at[k0]], ssa).wait()
        pltpu.async_copy(table.at[g_of(k0 + 3)], bufa, gsa)
        pltpu.make_async_copy(table.at[g_of(k0 + 2)], bufc, gsc).wait()
        pltpu.async_copy(bufc, acc_sh.at[sidx_v.at[k0 + 2]], ssc, add=True)
        pltpu.make_async_copy(bufb, acc_sh.at[sidx_v.at[k0 + 1]], ssb).wait()
        pltpu.async_copy(table.at[g_of(k0 + 4)], bufb, gsb)
        return 0
    lax.fori_loop(0, NCH // 3, body, 0)
    pltpu.make_async_copy(table.at[gidx_v.at[0]], bufa, gsa).wait()
    pltpu.make_async_copy(table.at[gidx_v.at[0]], bufb, gsb).wait()
    pltpu.make_async_copy(bufc, acc_sh.at[sidx_v.at[NCH - 1]], ssc).wait()

    plsc.subcore_barrier()

    eacc = bufa.at[pl.ds(0, EBLK)]
    escl = bufb.at[pl.ds(0, EBLK)]
    eres = bufc.at[pl.ds(0, EBLK)]

    def eblock(blk, _):
        r0 = s * ROWS + blk * EBLK
        pltpu.sync_copy(acc_sh.at[pl.ds(r0, EBLK)], eacc)
        pltpu.sync_copy(nsqb_hbm.at[d, pl.ds(r0, EBLK)], escl)

        def rows_g(r, _):
            for q in range(D // 16):
                sl = pl.ds(q * 16, 16)
                bufb[r, sl] = bufa[r, sl] * bufb[r, sl]
            return 0
        lax.fori_loop(0, EBLK, rows_g, 0)
        pltpu.sync_copy(escl, gnext_hbm.at[d, pl.ds(r0, EBLK)])

        pltpu.sync_copy(nb_hbm.at[d, pl.ds(r0, EBLK)], escl)
        pltpu.sync_copy(hsum_hbm.at[d, pl.ds(r0, EBLK)], eres)

        def rows_h(r, _):
            for q in range(D // 16):
                sl = pl.ds(q * 16, 16)
                bufc[r, sl] = bufc[r, sl] + bufa[r, sl] * bufb[r, sl]
            return 0
        lax.fori_loop(0, EBLK, rows_h, 0)
        pltpu.sync_copy(eres, hout_hbm.at[d, pl.ds(r0, EBLK)])
        return 0
    lax.fori_loop(0, ROWS // EBLK, eblock, 0)


@functools.partial(
    pl.kernel,
    out_type=jax.ShapeDtypeStruct((2, 2, P, D), jnp.float32),
    mesh=_mesh,
    scratch_types=[
        pltpu.VMEM((2 * (P // NT // SEC), SEC), jnp.int32),
        pltpu.VMEM((SEC, D), jnp.float32),
        pltpu.VMEM((SEC, D), jnp.float32),
        pltpu.VMEM((SEC, D), jnp.float32),
        pltpu.VMEM((SEC, D), jnp.float32),
        pltpu.SemaphoreType.DMA,
        pltpu.SemaphoreType.DMA,
        pltpu.SemaphoreType.DMA,
        pltpu.SemaphoreType.DMA,
        pltpu.SemaphoreType.DMA,
        pltpu.SemaphoreType.DMA,
        pltpu.SemaphoreType.DMA,
        pltpu.SemaphoreType.DMA,
    ],
)
def _score_gather_kernel(hsum_hbm, pidx_hbm, out_hbm, cidx_v, ba, bb, bc, bd,
                         ga, gb, gc, gd, wa, wb, wc, wd):
    c = lax.axis_index("c")
    s = lax.axis_index("s")
    ept = P // NT
    nch = ept // SEC
    tot = 2 * nch

    pltpu.sync_copy(pidx_hbm.at[c, 0, s], cidx_v.at[pl.ds(0, nch)])
    pltpu.sync_copy(pidx_hbm.at[c, 1, s], cidx_v.at[pl.ds(nch, nch)])

    def src(kk):
        kk = jnp.where(kk < tot, kk, 0)
        return hsum_hbm.at[kk // nch].at[cidx_v.at[kk]]

    def dst(kk):
        return out_hbm.at[c, kk // nch,
                          pl.ds(s * ept + lax.rem(kk, nch) * SEC, SEC)]

    pltpu.async_copy(src(0), ba, ga)
    pltpu.async_copy(src(1), bb, gb)
    pltpu.async_copy(src(2), bc, gc)

    def body(j, _):
        k0 = 4 * j
        pltpu.make_async_copy(src(k0), ba, ga).wait()
        pltpu.async_copy(ba, dst(k0), wa)

        @pl.when(j >= 1)
        def _():
            pltpu.make_async_copy(bd, dst(k0 - 1), wd).wait()
        pltpu.async_copy(src(k0 + 3), bd, gd)
        pltpu.make_async_copy(src(k0 + 1), bb, gb).wait()
        pltpu.async_copy(bb, dst(k0 + 1), wb)
        pltpu.make_async_copy(ba, dst(k0), wa).wait()
        pltpu.async_copy(src(k0 + 4), ba, ga)
        pltpu.make_async_copy(src(k0 + 2), bc, gc).wait()
        pltpu.async_copy(bc, dst(k0 + 2), wc)
        pltpu.make_async_copy(bb, dst(k0 + 1), wb).wait()
        pltpu.async_copy(src(k0 + 5), bb, gb)
        pltpu.make_async_copy(src(k0 + 3), bd, gd).wait()
        pltpu.async_copy(bd, dst(k0 + 3), wd)
        pltpu.make_async_copy(bc, dst(k0 + 2), wc).wait()
        pltpu.async_copy(src(k0 + 6), bc, gc)
        return 0
    lax.fori_loop(0, tot // 4, body, 0)
    pltpu.make_async_copy(src(0), ba, ga).wait()
    pltpu.make_async_copy(src(0), bb, gb).wait()
    pltpu.make_async_copy(src(0), bc, gc).wait()
    pltpu.make_async_copy(bd, dst(tot - 1), wd).wait()


def _prep_body(deg_ref, emb_ref, g0_ref, nsqb_ref, nb_ref):
    dg = jnp.maximum(deg_ref[0], 1.0)
    norm = lax.rsqrt(dg)
    nsq = 1.0 / dg
    g0_ref[0] = emb_ref[0] * norm
    nsqb_ref[0] = jnp.broadcast_to(nsq, nsqb_ref.shape[1:])
    nb_ref[0] = jnp.broadcast_to(norm, nb_ref.shape[1:])


_BR = 512


def _prep(deg3, empad):
    return pl.pallas_call(
        _prep_body,
        grid=(2, NP // _BR),
        in_specs=[
            pl.BlockSpec((1, _BR, 1), lambda i, j: (i, j, 0)),
            pl.BlockSpec((1, _BR, D), lambda i, j: (i, j, 0)),
        ],
        out_specs=[
            pl.BlockSpec((1, _BR, D), lambda i, j: (i, j, 0)),
            pl.BlockSpec((1, _BR, D), lambda i, j: (i, j, 0)),
            pl.BlockSpec((1, _BR, D), lambda i, j: (i, j, 0)),
        ],
        out_shape=[
            jax.ShapeDtypeStruct((2, NP, D), jnp.float32),
            jax.ShapeDtypeStruct((2, NP, D), jnp.float32),
            jax.ShapeDtypeStruct((2, NP, D), jnp.float32),
        ],
    )(deg3, empad)


def _dot_body(a_ref, b_ref, out_ref):
    out_ref[0] = (float(NLAYER) * float(NLAYER)) * jnp.sum(
        a_ref[0, 0] * b_ref[0, 0], axis=-1, keepdims=True)


_BP = 1024


def _dots(ab):
    return pl.pallas_call(
        _dot_body,
        grid=(2, P // _BP),
        in_specs=[
            pl.BlockSpec((1, 1, _BP, D), lambda i, j: (i, 0, j, 0)),
            pl.BlockSpec((1, 1, _BP, D), lambda i, j: (i, 1, j, 0)),
        ],
        out_specs=pl.BlockSpec((1, _BP, 1), lambda i, j: (i, j, 0)),
        out_shape=jax.ShapeDtypeStruct((2, P, 1), jnp.float32),
    )(ab, ab)


def kernel(msg_edges, pos_edges, neg_edges, user_emb, item_emb):
    pad_n = NT * NCH * EC - E
    pad_idx = U + (jnp.arange(pad_n, dtype=jnp.int32) % (NP - U))
    idx = jnp.concatenate(
        [msg_edges.astype(jnp.int32),
         jnp.broadcast_to(pad_idx, (2, pad_n))], axis=1)
    idx = idx.reshape(2, NT, NCH, EC)

    deg = _degree_kernel(idx)

    empad = jnp.stack([
        jnp.pad(user_emb, ((0, NP - U), (0, 0))),
        jnp.pad(item_emb, ((0, NP - U), (0, 0))),
    ])
    g0, nsqb, nb = _prep(deg.reshape(2, NP, 1), empad)

    zeros = jnp.zeros((NP, D), jnp.float32)
    g, hsum = g0, empad
    for _ in range(NLAYER):
        g, hsum = _layer_kernel(g, hsum, idx, nsqb, nb, zeros)

    pidx = jnp.stack([pos_edges, neg_edges]).astype(jnp.int32)
    pidx = pidx.reshape(2, 2, NT, P // NT // SEC, SEC)
    ab = _score_gather_kernel(hsum, pidx)
    scores = _dots(ab)
    return scores[0], scores[1]
